# baseline (speedup 1.0000x reference)
import functools

import jax
import jax.numpy as jnp
import numpy as np
from jax import lax
from jax.experimental import pallas as pl
from jax.experimental.pallas import tpu as pltpu
from jax.experimental.pallas import tpu_sc as plsc

N_NODES = 10000
D = 128
NC = 2
NS = 16
L = 16
NW = NC * NS
N_PAD = 10240
ROWS_PER_TILE = N_PAD // NS
CHUNK = 64
SEG = 8


def _unpack_perm():
    p = np.empty((D,), np.int64)
    for pos in range(D):
        k, j = divmod(pos, 32)
        p[pos] = 32 * k + (2 * j if j < 16 else 2 * (j - 16) + 1)
    return p


def _rsqrt16(x):
    i = plsc.bitcast(x, jnp.int32)
    y = plsc.bitcast(jnp.int32(0x5F3759DF) - (i >> 1), jnp.float32)
    for _ in range(3):
        y = y * (1.5 - 0.5 * x * y * y)
    return y


def _lane_bcast(v16, l):
    return v16.at[jnp.full((L,), l, jnp.int32)].get(mode="promise_in_bounds")


def _make_sc_kernel(e_rows):
    n_segs = e_rows // SEG
    deg_rows = (e_rows // (NS * SEG)) * SEG
    rem_blocks = (e_rows - deg_rows * NS) // SEG
    assert (e_rows - deg_rows * NS) % SEG == 0 and rem_blocks < NS
    mesh = plsc.VectorSubcoreMesh(core_axis_name="c", subcore_axis_name="s")

    @functools.partial(
        pl.kernel,
        out_type=jax.ShapeDtypeStruct((NC, N_PAD, D), jnp.float32),
        mesh=mesh,
        scratch_types=[
            pltpu.VMEM_SHARED((N_PAD, D), jnp.float32),
            pltpu.VMEM_SHARED((N_PAD,), jnp.float32),
            pltpu.VMEM((SEG, CHUNK), jnp.int32),
            pltpu.VMEM((SEG, CHUNK), jnp.int32),
            pltpu.VMEM((SEG, CHUNK), jnp.float32),
            pltpu.VMEM((SEG, CHUNK), jnp.float32),
            pltpu.VMEM((N_PAD,), jnp.float32),
            pltpu.VMEM((CHUNK, D // 2), jnp.int32),
            pltpu.VMEM((CHUNK, D // 2), jnp.int32),
            pltpu.VMEM((CHUNK, D), jnp.float32),
            pltpu.VMEM((CHUNK, D), jnp.float32),
            pltpu.VMEM((ROWS_PER_TILE,), jnp.float32),
            pltpu.SemaphoreType.DMA,
            pltpu.SemaphoreType.DMA,
            pltpu.SemaphoreType.DMA,
            pltpu.SemaphoreType.DMA,
            pltpu.SemaphoreType.DMA,
        ],
        compiler_params=pltpu.CompilerParams(
            needs_layout_passes=False, use_tc_tiling_on_sc=False),
    )
    def sc_kernel(src_hbm, dst_hbm, ew_hbm, xbf_hbm, agg_hbm,
                  out_sh, dinv_sh,
                  seg_src, seg_dst, seg_ew, seg_nrm,
                  dinv_loc, in_a, in_b, rows_a, rows_b, red_buf,
                  gsem_a, gsem_b, ssem_a, ssem_b, dsem):
        cid = lax.axis_index("c")
        sid = lax.axis_index("s")
        wid = sid * NC + cid
        zeros16 = jnp.zeros((L,), jnp.float32)

        def zrow(r, _):
            for k in range(D // L):
                rows_a[r, pl.ds(k * L, L)] = zeros16
            return 0
        lax.fori_loop(0, CHUNK, zrow, 0)

        def zred(i, _):
            red_buf[pl.ds(i * L, L)] = zeros16
            return 0
        lax.fori_loop(0, ROWS_PER_TILE // L, zred, 0)

        obase = sid * ROWS_PER_TILE
        pltpu.sync_copy(red_buf, dinv_sh.at[pl.ds(obase, ROWS_PER_TILE)])
        for t in range(ROWS_PER_TILE // CHUNK):
            pltpu.sync_copy(rows_a, out_sh.at[pl.ds(obase + t * CHUNK, CHUNK)])
        plsc.subcore_barrier()

        my_blocks = deg_rows // SEG
        dpairs = ((seg_src, seg_ew, gsem_a), (seg_dst, seg_nrm, gsem_b))

        def fire_loads(c):
            ib, fb, gs = dpairs[c % 2]
            row0 = sid * deg_rows + c * SEG
            return (pltpu.async_copy(dst_hbm.at[pl.ds(row0, SEG)], ib, gs),
                    pltpu.async_copy(ew_hbm.at[pl.ds(row0, SEG)], fb, dsem))

        pend = fire_loads(0)
        prev_adds = []
        for c in range(my_blocks):
            ib, fb, _ = dpairs[c % 2]
            pend[0].wait()
            pend[1].wait()
            adds = [pltpu.async_copy(fb.at[r], dinv_sh.at[ib.at[r]],
                                     ssem_a, add=True)
                    for r in range(SEG)]
            for d in prev_adds:
                d.wait()
            if c + 1 < my_blocks:
                pend = fire_loads(c + 1)
            prev_adds = adds
        for d in prev_adds:
            d.wait()
        if rem_blocks:
            @pl.when(sid < rem_blocks)
            def _():
                row0 = NS * deg_rows + sid * SEG
                pltpu.sync_copy(dst_hbm.at[pl.ds(row0, SEG)], seg_src)
                pltpu.sync_copy(ew_hbm.at[pl.ds(row0, SEG)], seg_ew)
                rds = [pltpu.async_copy(seg_ew.at[r],
                                        dinv_sh.at[seg_src.at[r]],
                                        ssem_a, add=True)
                       for r in range(SEG)]
                for d in rds:
                    d.wait()
        plsc.subcore_barrier()

        pltpu.sync_copy(dinv_sh.at[pl.ds(obase, ROWS_PER_TILE)], red_buf)

        def dinv_vec(i, _):
            sl = pl.ds(i * L, L)
            red_buf[sl] = _rsqrt16(red_buf[sl])
            return 0
        lax.fori_loop(0, ROWS_PER_TILE // L, dinv_vec, 0)
        pltpu.sync_copy(red_buf, dinv_sh.at[pl.ds(obase, ROWS_PER_TILE)])
        plsc.subcore_barrier()
        pltpu.sync_copy(dinv_sh, dinv_loc)

        n_my_segs = (n_segs - wid + NW - 1) // NW
        ibufs = (in_a, in_b)
        obufs = (rows_a, rows_b)
        gsems = (gsem_a, gsem_b)
        ssems = (ssem_a, ssem_b)

        def scale_chunk(r, ibuf, obuf):
            def scale_g(g, _):
                n16 = seg_nrm[r, pl.ds(g * L, L)]
                for l in range(L):
                    spl = _lane_bcast(n16, l)
                    j = g * L + l
                    for k in range(D // 32):
                        vi = ibuf[j, pl.ds(k * L, L)]
                        v = plsc.bitcast(vi, jnp.bfloat16)
                        a, bb = plsc.unpack(v, format=plsc.PackFormat.INTERLEAVED)
                        obuf[j, pl.ds(k * 32, L)] = a * spl
                        obuf[j, pl.ds(k * 32 + L, L)] = bb * spl
                return 0
            lax.fori_loop(0, CHUNK // L, scale_g, 0)

        def seg_body(t, _):
            segbase = (wid + t * NW) * SEG
            pltpu.sync_copy(src_hbm.at[pl.ds(segbase, SEG)], seg_src)
            pltpu.sync_copy(dst_hbm.at[pl.ds(segbase, SEG)], seg_dst)
            pltpu.sync_copy(ew_hbm.at[pl.ds(segbase, SEG)], seg_ew)

            gd = {0: pltpu.async_copy(xbf_hbm.at[seg_src.at[0]], in_a, gsem_a)}

            def norm_row(r, _):
                for k in range(CHUNK // L):
                    sl = pl.ds(k * L, L)
                    s16 = seg_src[r, sl]
                    d16 = seg_dst[r, sl]
                    seg_nrm[r, sl] = (plsc.load_gather(dinv_loc, [s16])
                                      * seg_ew[r, sl]
                                      * plsc.load_gather(dinv_loc, [d16]))
                return 0
            lax.fori_loop(0, SEG, norm_row, 0)

            sd = {}
            for r in range(SEG):
                p = r % 2
                if r + 1 < SEG:
                    if r - 1 >= 0:
                        sd[r - 1].wait()
                    gd[r + 1] = pltpu.async_copy(
                        xbf_hbm.at[seg_src.at[r + 1]], ibufs[1 - p],
                        gsems[1 - p])
                gd[r].wait()
                scale_chunk(r, ibufs[p], obufs[p])
                sd[r] = pltpu.async_copy(
                    obufs[p], out_sh.at[seg_dst.at[r]], ssems[p], add=True)
            sd[SEG - 2].wait()
            sd[SEG - 1].wait()
            return 0
        lax.fori_loop(0, n_my_segs, seg_body, 0)
        plsc.subcore_barrier()

        for t in range(ROWS_PER_TILE // CHUNK):
            r0 = obase + t * CHUNK
            pltpu.sync_copy(out_sh.at[pl.ds(r0, CHUNK)],
                            agg_hbm.at[cid, pl.ds(r0, CHUNK)])

    return sc_kernel


def _tc_body(a_ref, w_ref, b_ref, o_ref):
    a = a_ref[0] + a_ref[1]
    h = jnp.dot(a, w_ref[...], preferred_element_type=jnp.float32)
    o_ref[...] = jnp.maximum(h + b_ref[...], 0.0)


def _tc_finish(agg, W, b2d):
    bm = 1024
    return pl.pallas_call(
        _tc_body,
        grid=(N_PAD // bm,),
        in_specs=[
            pl.BlockSpec((NC, bm, D), lambda i: (0, i, 0)),
            pl.BlockSpec((D, D), lambda i: (0, 0)),
            pl.BlockSpec((1, D), lambda i: (0, 0)),
        ],
        out_specs=pl.BlockSpec((bm, D), lambda i: (i, 0)),
        out_shape=jax.ShapeDtypeStruct((N_PAD, D), jnp.float32),
    )(agg, W, b2d)


def kernel(x, edge_index, w, W, b):
    N = x.shape[0]
    E = edge_index.shape[1]
    src = edge_index[0].astype(jnp.int32)
    dst = edge_index[1].astype(jnp.int32)
    loop = jnp.arange(N, dtype=jnp.int32)
    e_tot = E + N
    e_pad = ((e_tot + NW * 128 - 1) // (NW * 128)) * (NW * 128)
    pad = e_pad - e_tot
    pad_idx = (jnp.arange(pad, dtype=jnp.int32) * 97) % N
    src_all = jnp.concatenate([src, loop, pad_idx]).reshape(e_pad // CHUNK, CHUNK)
    dst_all = jnp.concatenate([dst, loop, pad_idx]).reshape(e_pad // CHUNK, CHUNK)
    ew_all = jnp.concatenate(
        [w, jnp.ones((N,), w.dtype), jnp.zeros((pad,), w.dtype)]
    ).reshape(e_pad // CHUNK, CHUNK)
    xbf = jnp.concatenate(
        [x, jnp.zeros((N_PAD - N, D), x.dtype)], axis=0).astype(jnp.bfloat16)
    xi32 = lax.bitcast_convert_type(
        xbf.reshape(N_PAD, D // 2, 2), jnp.int32)
    W_perm = W[_unpack_perm()]

    agg = _make_sc_kernel(e_pad // CHUNK)(src_all, dst_all, ew_all, xi32)
    out = _tc_finish(agg, W_perm, b.reshape(1, D))
    return out[:N]


# speedup vs baseline: 2.0409x; 2.0409x over previous
import functools

import jax
import jax.numpy as jnp
from jax import lax
from jax.experimental import pallas as pl
from jax.experimental.pallas import tpu as pltpu
from jax.experimental.pallas import tpu_sc as plsc

N_NODES = 10000
D = 128
NC = 2
NS = 16
L = 16
NW = NC * NS
N_PAD = 10240
ROWS_PER_TILE = N_PAD // NS
CHUNK = 128
SEG = 8


def _rsqrt16(x):
    i = plsc.bitcast(x, jnp.int32)
    y = plsc.bitcast(jnp.int32(0x5F3759DF) - (i >> 1), jnp.float32)
    for _ in range(3):
        y = y * (1.5 - 0.5 * x * y * y)
    return y


def _make_sc_kernel(e_rows):
    n_segs = e_rows // SEG
    deg_rows = (e_rows // (NS * SEG)) * SEG
    rem_blocks = (e_rows - deg_rows * NS) // SEG
    assert (e_rows - deg_rows * NS) % SEG == 0 and rem_blocks < NS
    mesh = plsc.VectorSubcoreMesh(core_axis_name="c", subcore_axis_name="s")

    @functools.partial(
        pl.kernel,
        out_type=jax.ShapeDtypeStruct((NC, N_PAD, D), jnp.float32),
        mesh=mesh,
        scratch_types=[
            pltpu.VMEM_SHARED((N_PAD, D), jnp.float32),
            pltpu.VMEM_SHARED((N_PAD,), jnp.float32),
            pltpu.VMEM((SEG, CHUNK), jnp.int32),
            pltpu.VMEM((SEG, CHUNK), jnp.int32),
            pltpu.VMEM((SEG, CHUNK), jnp.float32),
            pltpu.VMEM((SEG, CHUNK), jnp.float32),
            pltpu.VMEM((N_PAD,), jnp.float32),
            pltpu.VMEM((CHUNK, D), jnp.float32),
            pltpu.VMEM((CHUNK, D), jnp.float32),
            pltpu.VMEM((ROWS_PER_TILE,), jnp.float32),
            pltpu.SemaphoreType.DMA,
            pltpu.SemaphoreType.DMA,
            pltpu.SemaphoreType.DMA,
            pltpu.SemaphoreType.DMA,
            pltpu.SemaphoreType.DMA,
        ],
        compiler_params=pltpu.CompilerParams(needs_layout_passes=False),
    )
    def sc_kernel(src_hbm, dst_hbm, ew_hbm, x_hbm, agg_hbm,
                  out_sh, dinv_sh,
                  seg_src, seg_dst, seg_ew, seg_nrm,
                  dinv_loc, rows_a, rows_b, red_buf,
                  gsem_a, gsem_b, ssem_a, ssem_b, dsem):
        cid = lax.axis_index("c")
        sid = lax.axis_index("s")
        wid = sid * NC + cid
        zeros16 = jnp.zeros((L,), jnp.float32)

        def zrow(r, _):
            for k in range(D // L):
                rows_a[r, pl.ds(k * L, L)] = zeros16
            return 0
        lax.fori_loop(0, CHUNK, zrow, 0)

        def zred(i, _):
            red_buf[pl.ds(i * L, L)] = zeros16
            return 0
        lax.fori_loop(0, ROWS_PER_TILE // L, zred, 0)

        obase = sid * ROWS_PER_TILE
        zds = [pltpu.async_copy(red_buf,
                                dinv_sh.at[pl.ds(obase, ROWS_PER_TILE)],
                                dsem)]
        for t in range(ROWS_PER_TILE // CHUNK):
            zds.append(pltpu.async_copy(
                rows_a, out_sh.at[pl.ds(obase + t * CHUNK, CHUNK)], dsem))
        for d in zds:
            d.wait()
        plsc.subcore_barrier()

        my_blocks = deg_rows // SEG
        dpairs = ((seg_src, seg_ew, gsem_a), (seg_dst, seg_nrm, gsem_b))

        def fire_loads(c):
            ib, fb, gs = dpairs[c % 2]
            row0 = sid * deg_rows + c * SEG
            return (pltpu.async_copy(dst_hbm.at[pl.ds(row0, SEG)], ib, gs),
                    pltpu.async_copy(ew_hbm.at[pl.ds(row0, SEG)], fb, dsem))

        pend = fire_loads(0)
        prev_adds = []
        for c in range(my_blocks):
            ib, fb, _ = dpairs[c % 2]
            pend[0].wait()
            pend[1].wait()
            adds = [pltpu.async_copy(fb.at[r], dinv_sh.at[ib.at[r]],
                                     ssem_a, add=True)
                    for r in range(SEG)]
            for d in prev_adds:
                d.wait()
            if c + 1 < my_blocks:
                pend = fire_loads(c + 1)
            prev_adds = adds
        for d in prev_adds:
            d.wait()
        if rem_blocks:
            @pl.when(sid < rem_blocks)
            def _():
                row0 = NS * deg_rows + sid * SEG
                pltpu.sync_copy(dst_hbm.at[pl.ds(row0, SEG)], seg_src)
                pltpu.sync_copy(ew_hbm.at[pl.ds(row0, SEG)], seg_ew)
                rds = [pltpu.async_copy(seg_ew.at[r],
                                        dinv_sh.at[seg_src.at[r]],
                                        ssem_a, add=True)
                       for r in range(SEG)]
                for d in rds:
                    d.wait()
        plsc.subcore_barrier()

        pltpu.sync_copy(dinv_sh.at[pl.ds(obase, ROWS_PER_TILE)], red_buf)

        def dinv_vec(i, _):
            sl = pl.ds(i * L, L)
            red_buf[sl] = _rsqrt16(red_buf[sl])
            return 0
        lax.fori_loop(0, ROWS_PER_TILE // L, dinv_vec, 0)
        pltpu.sync_copy(red_buf, dinv_sh.at[pl.ds(obase, ROWS_PER_TILE)])
        plsc.subcore_barrier()
        pltpu.sync_copy(dinv_sh, dinv_loc)

        n_my_segs = (n_segs - wid + NW - 1) // NW

        bufs = (rows_a, rows_b)
        gsems = (gsem_a, gsem_b)
        ssems = (ssem_a, ssem_b)

        def scale_chunk(r, buf):
            def scale_g(g, _):
                n16 = seg_nrm[r, pl.ds(g * L, L)]
                for l in range(L):
                    nspl = n16.at[jnp.full((L,), l, jnp.int32)].get(
                        mode="promise_in_bounds")
                    for k in range(D // L):
                        sl = pl.ds(k * L, L)
                        buf[g * L + l, sl] = buf[g * L + l, sl] * nspl
                return 0
            lax.fori_loop(0, CHUNK // L, scale_g, 0)

        def seg_body(t, _):
            segbase = (wid + t * NW) * SEG
            lds = (pltpu.async_copy(src_hbm.at[pl.ds(segbase, SEG)],
                                    seg_src, dsem),
                   pltpu.async_copy(dst_hbm.at[pl.ds(segbase, SEG)],
                                    seg_dst, dsem),
                   pltpu.async_copy(ew_hbm.at[pl.ds(segbase, SEG)],
                                    seg_ew, dsem))
            for d in lds:
                d.wait()

            gd = {0: pltpu.async_copy(x_hbm.at[seg_src.at[0]], rows_a, gsem_a)}

            def norm_row(r, _):
                for k in range(D // L):
                    sl = pl.ds(k * L, L)
                    s16 = seg_src[r, sl]
                    d16 = seg_dst[r, sl]
                    seg_nrm[r, sl] = (plsc.load_gather(dinv_loc, [s16])
                                      * seg_ew[r, sl]
                                      * plsc.load_gather(dinv_loc, [d16]))
                return 0
            lax.fori_loop(0, SEG, norm_row, 0)

            sd = {}
            for r in range(SEG):
                p = r % 2
                if r + 1 < SEG:
                    if r - 1 >= 0:
                        sd[r - 1].wait()
                    gd[r + 1] = pltpu.async_copy(
                        x_hbm.at[seg_src.at[r + 1]], bufs[1 - p],
                        gsems[1 - p])
                gd[r].wait()
                scale_chunk(r, bufs[p])
                sd[r] = pltpu.async_copy(
                    bufs[p], out_sh.at[seg_dst.at[r]], ssems[p], add=True)
            sd[SEG - 2].wait()
            sd[SEG - 1].wait()
            return 0
        lax.fori_loop(0, n_my_segs, seg_body, 0)
        plsc.subcore_barrier()

        wds = []
        for t in range(ROWS_PER_TILE // CHUNK):
            r0 = obase + t * CHUNK
            wds.append(pltpu.async_copy(out_sh.at[pl.ds(r0, CHUNK)],
                                        agg_hbm.at[cid, pl.ds(r0, CHUNK)],
                                        dsem))
        for d in wds:
            d.wait()

    return sc_kernel


def _tc_body(a_ref, w_ref, b_ref, o_ref):
    a = a_ref[0] + a_ref[1]
    h = jnp.dot(a, w_ref[...], preferred_element_type=jnp.float32)
    o_ref[...] = jnp.maximum(h + b_ref[...], 0.0)


def _tc_finish(agg, W, b2d):
    bm = 1024
    return pl.pallas_call(
        _tc_body,
        grid=(N_PAD // bm,),
        in_specs=[
            pl.BlockSpec((NC, bm, D), lambda i: (0, i, 0)),
            pl.BlockSpec((D, D), lambda i: (0, 0)),
            pl.BlockSpec((1, D), lambda i: (0, 0)),
        ],
        out_specs=pl.BlockSpec((bm, D), lambda i: (i, 0)),
        out_shape=jax.ShapeDtypeStruct((N_PAD, D), jnp.float32),
    )(agg, W, b2d)


def kernel(x, edge_index, w, W, b):
    N = x.shape[0]
    E = edge_index.shape[1]
    src = edge_index[0].astype(jnp.int32)
    dst = edge_index[1].astype(jnp.int32)
    loop = jnp.arange(N, dtype=jnp.int32)
    e_tot = E + N
    e_pad = ((e_tot + NW * CHUNK - 1) // (NW * CHUNK)) * (NW * CHUNK)
    pad = e_pad - e_tot
    pad_idx = (jnp.arange(pad, dtype=jnp.int32) * 97) % N
    src_all = jnp.concatenate([src, loop, pad_idx]).reshape(e_pad // CHUNK, CHUNK)
    dst_all = jnp.concatenate([dst, loop, pad_idx]).reshape(e_pad // CHUNK, CHUNK)
    ew_all = jnp.concatenate(
        [w, jnp.ones((N,), w.dtype), jnp.zeros((pad,), w.dtype)]
    ).reshape(e_pad // CHUNK, CHUNK)

    agg = _make_sc_kernel(e_pad // CHUNK)(src_all, dst_all, ew_all, x)
    out = _tc_finish(agg, W, b.reshape(1, D))
    return out[:N]


# speedup vs baseline: 2.0552x; 1.0070x over previous
import functools

import jax
import jax.numpy as jnp
from jax import lax
from jax.experimental import pallas as pl
from jax.experimental.pallas import tpu as pltpu
from jax.experimental.pallas import tpu_sc as plsc

N_NODES = 10000
D = 128
NC = 2
NS = 16
L = 16
NW = NC * NS
N_PAD = 10240
ROWS_PER_TILE = N_PAD // NS
CHUNK = 128
SEG = 8


def _rsqrt16(x):
    i = plsc.bitcast(x, jnp.int32)
    y = plsc.bitcast(jnp.int32(0x5F3759DF) - (i >> 1), jnp.float32)
    for _ in range(3):
        y = y * (1.5 - 0.5 * x * y * y)
    return y


def _make_sc_kernel(e_rows):
    n_segs = e_rows // SEG
    deg_rows = (e_rows // (NS * SEG)) * SEG
    rem_blocks = (e_rows - deg_rows * NS) // SEG
    assert (e_rows - deg_rows * NS) % SEG == 0 and rem_blocks < NS
    mesh = plsc.VectorSubcoreMesh(core_axis_name="c", subcore_axis_name="s")

    @functools.partial(
        pl.kernel,
        out_type=jax.ShapeDtypeStruct((NC, N_PAD, D), jnp.float32),
        mesh=mesh,
        scratch_types=[
            pltpu.VMEM_SHARED((N_PAD, D), jnp.float32),
            pltpu.VMEM_SHARED((N_PAD,), jnp.float32),
            pltpu.VMEM((SEG, CHUNK), jnp.int32),
            pltpu.VMEM((SEG, CHUNK), jnp.int32),
            pltpu.VMEM((SEG, CHUNK), jnp.float32),
            pltpu.VMEM((SEG, CHUNK), jnp.float32),
            pltpu.VMEM((N_PAD,), jnp.float32),
            pltpu.VMEM((CHUNK, D), jnp.float32),
            pltpu.VMEM((CHUNK, D), jnp.float32),
            pltpu.VMEM((ROWS_PER_TILE,), jnp.float32),
            pltpu.SemaphoreType.DMA,
            pltpu.SemaphoreType.DMA,
            pltpu.SemaphoreType.DMA,
            pltpu.SemaphoreType.DMA,
            pltpu.SemaphoreType.DMA,
        ],
        compiler_params=pltpu.CompilerParams(needs_layout_passes=False),
    )
    def sc_kernel(src_hbm, dst_hbm, ew_hbm, x_hbm, agg_hbm,
                  out_sh, dinv_sh,
                  seg_src, seg_dst, seg_ew, seg_nrm,
                  dinv_loc, rows_a, rows_b, red_buf,
                  gsem_a, gsem_b, ssem_a, ssem_b, dsem):
        cid = lax.axis_index("c")
        sid = lax.axis_index("s")
        wid = sid * NC + cid
        zeros16 = jnp.zeros((L,), jnp.float32)

        def zrow(r, _):
            for k in range(D // L):
                rows_a[r, pl.ds(k * L, L)] = zeros16
            return 0
        lax.fori_loop(0, CHUNK, zrow, 0)

        def zred(i, _):
            red_buf[pl.ds(i * L, L)] = zeros16
            return 0
        lax.fori_loop(0, ROWS_PER_TILE // L, zred, 0)

        obase = sid * ROWS_PER_TILE
        zds = [pltpu.async_copy(red_buf,
                                dinv_sh.at[pl.ds(obase, ROWS_PER_TILE)],
                                dsem)]
        for t in range(ROWS_PER_TILE // CHUNK):
            zds.append(pltpu.async_copy(
                rows_a, out_sh.at[pl.ds(obase + t * CHUNK, CHUNK)], dsem))
        for d in zds:
            d.wait()
        plsc.subcore_barrier()

        my_blocks = deg_rows // SEG
        dpairs = ((seg_src, seg_ew, gsem_a), (seg_dst, seg_nrm, gsem_b))

        def fire_loads(c):
            ib, fb, gs = dpairs[c % 2]
            row0 = sid * deg_rows + c * SEG
            return (pltpu.async_copy(dst_hbm.at[pl.ds(row0, SEG)], ib, gs),
                    pltpu.async_copy(ew_hbm.at[pl.ds(row0, SEG)], fb, dsem))

        pend = fire_loads(0)
        prev_adds = []
        for c in range(my_blocks):
            ib, fb, _ = dpairs[c % 2]
            pend[0].wait()
            pend[1].wait()
            adds = [pltpu.async_copy(fb.at[r], dinv_sh.at[ib.at[r]],
                                     ssem_a, add=True)
                    for r in range(SEG)]
            for d in prev_adds:
                d.wait()
            if c + 1 < my_blocks:
                pend = fire_loads(c + 1)
            prev_adds = adds
        for d in prev_adds:
            d.wait()
        if rem_blocks:
            @pl.when(sid >= NS - rem_blocks)
            def _():
                row0 = NS * deg_rows + (sid - (NS - rem_blocks)) * SEG
                pltpu.sync_copy(dst_hbm.at[pl.ds(row0, SEG)], seg_src)
                pltpu.sync_copy(ew_hbm.at[pl.ds(row0, SEG)], seg_ew)
                rds = [pltpu.async_copy(seg_ew.at[r],
                                        dinv_sh.at[seg_src.at[r]],
                                        ssem_a, add=True)
                       for r in range(SEG)]
                for d in rds:
                    d.wait()
        plsc.subcore_barrier()

        pltpu.sync_copy(dinv_sh.at[pl.ds(obase, ROWS_PER_TILE)], red_buf)

        def dinv_vec(i, _):
            sl = pl.ds(i * L, L)
            red_buf[sl] = _rsqrt16(red_buf[sl])
            return 0
        lax.fori_loop(0, ROWS_PER_TILE // L, dinv_vec, 0)
        pltpu.sync_copy(red_buf, dinv_sh.at[pl.ds(obase, ROWS_PER_TILE)])
        plsc.subcore_barrier()
        pltpu.sync_copy(dinv_sh, dinv_loc)

        n_my_segs = (n_segs - wid + NW - 1) // NW

        bufs = (rows_a, rows_b)
        gsems = (gsem_a, gsem_b)
        ssems = (ssem_a, ssem_b)

        def scale_chunk(r, buf):
            def scale_g(g, _):
                n16 = seg_nrm[r, pl.ds(g * L, L)]
                for l in range(L):
                    nspl = n16.at[jnp.full((L,), l, jnp.int32)].get(
                        mode="promise_in_bounds")
                    for k in range(D // L):
                        sl = pl.ds(k * L, L)
                        buf[g * L + l, sl] = buf[g * L + l, sl] * nspl
                return 0
            lax.fori_loop(0, CHUNK // L, scale_g, 0)

        def seg_body(t, _):
            segbase = (wid + t * NW) * SEG
            lds = (pltpu.async_copy(src_hbm.at[pl.ds(segbase, SEG)],
                                    seg_src, dsem),
                   pltpu.async_copy(dst_hbm.at[pl.ds(segbase, SEG)],
                                    seg_dst, dsem),
                   pltpu.async_copy(ew_hbm.at[pl.ds(segbase, SEG)],
                                    seg_ew, dsem))
            for d in lds:
                d.wait()

            gd = {0: pltpu.async_copy(x_hbm.at[seg_src.at[0]], rows_a, gsem_a)}

            def norm_row(r, _):
                for k in range(D // L):
                    sl = pl.ds(k * L, L)
                    s16 = seg_src[r, sl]
                    d16 = seg_dst[r, sl]
                    seg_nrm[r, sl] = (plsc.load_gather(dinv_loc, [s16])
                                      * seg_ew[r, sl]
                                      * plsc.load_gather(dinv_loc, [d16]))
                return 0
            lax.fori_loop(0, SEG, norm_row, 0)

            sd = {}
            for r in range(SEG):
                p = r % 2
                if r + 1 < SEG:
                    if r - 1 >= 0:
                        sd[r - 1].wait()
                    gd[r + 1] = pltpu.async_copy(
                        x_hbm.at[seg_src.at[r + 1]], bufs[1 - p],
                        gsems[1 - p])
                gd[r].wait()
                scale_chunk(r, bufs[p])
                sd[r] = pltpu.async_copy(
                    bufs[p], out_sh.at[seg_dst.at[r]], ssems[p], add=True)
            sd[SEG - 2].wait()
            sd[SEG - 1].wait()
            return 0
        lax.fori_loop(0, n_my_segs, seg_body, 0)
        plsc.subcore_barrier()

        wds = []
        for t in range(ROWS_PER_TILE // CHUNK):
            r0 = obase + t * CHUNK
            wds.append(pltpu.async_copy(out_sh.at[pl.ds(r0, CHUNK)],
                                        agg_hbm.at[cid, pl.ds(r0, CHUNK)],
                                        dsem))
        for d in wds:
            d.wait()

    return sc_kernel


def _tc_body(a_ref, w_ref, b_ref, o_ref):
    a = a_ref[0] + a_ref[1]
    h = jnp.dot(a, w_ref[...], preferred_element_type=jnp.float32)
    o_ref[...] = jnp.maximum(h + b_ref[...], 0.0)


def _tc_finish(agg, W, b2d):
    bm = 1024
    return pl.pallas_call(
        _tc_body,
        grid=(N_PAD // bm,),
        in_specs=[
            pl.BlockSpec((NC, bm, D), lambda i: (0, i, 0)),
            pl.BlockSpec((D, D), lambda i: (0, 0)),
            pl.BlockSpec((1, D), lambda i: (0, 0)),
        ],
        out_specs=pl.BlockSpec((bm, D), lambda i: (i, 0)),
        out_shape=jax.ShapeDtypeStruct((N_PAD, D), jnp.float32),
    )(agg, W, b2d)


def kernel(x, edge_index, w, W, b):
    N = x.shape[0]
    E = edge_index.shape[1]
    src = edge_index[0].astype(jnp.int32)
    dst = edge_index[1].astype(jnp.int32)
    loop = jnp.arange(N, dtype=jnp.int32)
    e_tot = E + N
    e_pad = ((e_tot + NW * CHUNK - 1) // (NW * CHUNK)) * (NW * CHUNK)
    pad = e_pad - e_tot
    pad_idx = (jnp.arange(pad, dtype=jnp.int32) * 97) % N
    src_all = jnp.concatenate([src, loop, pad_idx]).reshape(e_pad // CHUNK, CHUNK)
    dst_all = jnp.concatenate([dst, loop, pad_idx]).reshape(e_pad // CHUNK, CHUNK)
    ew_all = jnp.concatenate(
        [w, jnp.ones((N,), w.dtype), jnp.zeros((pad,), w.dtype)]
    ).reshape(e_pad // CHUNK, CHUNK)

    agg = _make_sc_kernel(e_pad // CHUNK)(src_all, dst_all, ew_all, x)
    out = _tc_finish(agg, W, b.reshape(1, D))
    return out[:N]
